# baseline (speedup 1.0000x reference)
import jax
import jax.numpy as jnp
from jax.experimental import pallas as pl
from jax.experimental.pallas import tpu as pltpu

_H = 12
_S = 2048
_D = 768
_BLK = 512


def _colsum_body(p_ref, out_ref):
    r = pl.program_id(1)

    @pl.when(r == 0)
    def _():
        out_ref[...] = jnp.zeros_like(out_ref)

    out_ref[...] += jnp.sum(p_ref[...], axis=1, keepdims=True)


def _finalize_body(h_ref, cs_ref, hm_ref, thr_ref, temp_ref, out_ref):
    hm = hm_ref[...]
    imp = jax.lax.dot_general(
        cs_ref[...], hm,
        dimension_numbers=(((0,), (0,)), ((), ())),
        preferred_element_type=jnp.float32,
    )
    imp = imp / jnp.sum(hm)
    row = jax.lax.broadcasted_iota(jnp.int32, imp.shape, 0)
    imp = jnp.where(row == 0, imp + 100.0, imp)
    mask = jax.nn.sigmoid((imp - thr_ref[0, 0]) / temp_ref[0, 0])
    out_ref[...] = h_ref[...] * mask


def kernel(hidden_states, attention_probs, head_masks, attention_mask, temp, threshold):
    probs = attention_probs.reshape(_H, _S, _S)

    colsums = pl.pallas_call(
        _colsum_body,
        grid=(_H, _S // _BLK),
        in_specs=[pl.BlockSpec((1, _BLK, _S), lambda h, r: (h, r, 0))],
        out_specs=pl.BlockSpec((1, 1, _S), lambda h, r: (h, 0, 0)),
        out_shape=jax.ShapeDtypeStruct((_H, 1, _S), jnp.float32),
        compiler_params=pltpu.CompilerParams(
            dimension_semantics=("parallel", "arbitrary"),
        ),
    )(probs)

    hidden = hidden_states.reshape(_S, _D)
    out = pl.pallas_call(
        _finalize_body,
        in_specs=[
            pl.BlockSpec((_S, _D), lambda: (0, 0)),
            pl.BlockSpec((_H, _S), lambda: (0, 0)),
            pl.BlockSpec((_H, 1), lambda: (0, 0)),
            pl.BlockSpec((1, 1), lambda: (0, 0)),
            pl.BlockSpec((1, 1), lambda: (0, 0)),
        ],
        out_specs=pl.BlockSpec((_S, _D), lambda: (0, 0)),
        out_shape=jax.ShapeDtypeStruct((_S, _D), jnp.float32),
    )(hidden, colsums.reshape(_H, _S), head_masks.reshape(_H, 1),
      threshold.reshape(1, 1), temp.reshape(1, 1))

    return (out.reshape(1, _S, _D), attention_mask)
